# Initial kernel scaffold; baseline (speedup 1.0000x reference)
import jax
import jax.numpy as jnp
from jax.experimental import pallas as pl


def kernel(tokens, table):
    raise NotImplementedError("write your pallas kernel here")


# speedup vs baseline: 2.4196x; 2.4196x over previous
import functools
import math

import jax
import jax.numpy as jnp
from jax import lax
from jax.experimental import pallas as pl
from jax.experimental.pallas import tpu as pltpu
from jax.experimental.pallas import tpu_sc as plsc

_EMB = 128
_SCALE = math.sqrt(_EMB)
_NC = 2
_NS = 16
_NW = _NC * _NS
_C = 128
_LANES = 16


def _emb_body(tok_hbm, table_hbm, out_hbm, idx_v, rows_v, s_in, s_out):
    wid = lax.axis_index("s") * _NC + lax.axis_index("c")
    n_chunks = idx_v.shape[0]
    base = wid * (n_chunks * _C)

    pltpu.sync_copy(tok_hbm.at[wid], idx_v)

    def step(g, carry):
        pltpu.async_copy(table_hbm.at[idx_v.at[g]], rows_v, s_in).wait()

        def row(i, c):
            for j in range(_EMB // _LANES):
                sl = pl.ds(j * _LANES, _LANES)
                rows_v[i, sl] = rows_v[i, sl] * _SCALE
            return c

        lax.fori_loop(0, _C, row, 0)
        pltpu.async_copy(rows_v, out_hbm.at[pl.ds(base + g * _C, _C)], s_out).wait()
        return carry

    lax.fori_loop(0, n_chunks, step, 0)


def kernel(tokens, table):
    b, l = tokens.shape
    n = b * l
    assert n % (_NW * _C) == 0
    n_chunks = n // (_NW * _C)
    tok3 = tokens.reshape(_NW, n_chunks, _C).astype(jnp.int32)

    grid_kernel = functools.partial(
        pl.kernel,
        mesh=plsc.VectorSubcoreMesh(core_axis_name="c", subcore_axis_name="s"),
        out_type=jax.ShapeDtypeStruct((n, _EMB), jnp.float32),
        scratch_types=[
            pltpu.VMEM((n_chunks, _C), jnp.int32),
            pltpu.VMEM((_C, _EMB), jnp.float32),
            pltpu.SemaphoreType.DMA,
            pltpu.SemaphoreType.DMA,
        ],
    )(_emb_body)

    out = grid_kernel(tok3, table)
    return out.reshape(b, l, _EMB)


# speedup vs baseline: 2.7319x; 1.1291x over previous
import functools
import math

import jax
import jax.numpy as jnp
from jax import lax
from jax.experimental import pallas as pl
from jax.experimental.pallas import tpu as pltpu
from jax.experimental.pallas import tpu_sc as plsc

_EMB = 128
_SCALE = math.sqrt(_EMB)
_NC = 2
_NS = 16
_NW = _NC * _NS
_C = 128
_LANES = 16
_K = 5


def _emb_body(tok_hbm, table_hbm, out_hbm, idx_v, rows0, rows1, s_in0, s_in1,
              s_out0, s_out1):
    wid = lax.axis_index("s") * _NC + lax.axis_index("c")
    n_chunks = idx_v.shape[0]
    base = wid * (n_chunks * _C)

    pltpu.sync_copy(tok_hbm.at[wid], idx_v)

    def scale(rows):
        def row(i, c):
            for j in range(_EMB // _LANES):
                sl = pl.ds(j * _LANES, _LANES)
                rows[i, sl] = rows[i, sl] * _SCALE
            return c

        lax.fori_loop(0, _C, row, 0)

    def outer(t, carry):
        a = 2 * t
        ga = pltpu.async_copy(table_hbm.at[idx_v.at[a]], rows0, s_in0)
        gb = pltpu.async_copy(table_hbm.at[idx_v.at[a + 1]], rows1, s_in1)
        ga.wait()
        scale(rows0)
        oa = pltpu.async_copy(rows0, out_hbm.at[pl.ds(base + a * _C, _C)], s_out0)
        gb.wait()
        scale(rows1)
        ob = pltpu.async_copy(rows1, out_hbm.at[pl.ds(base + (a + 1) * _C, _C)], s_out1)
        oa.wait()
        ob.wait()
        return carry

    lax.fori_loop(0, n_chunks // 2, outer, 0)


def kernel(tokens, table):
    b, l = tokens.shape
    n = b * l
    assert n % (_NW * _C) == 0
    n_chunks = n // (_NW * _C)
    assert n_chunks % 2 == 0
    tok3 = tokens.reshape(_NW, n_chunks, _C).astype(jnp.int32)

    grid_kernel = functools.partial(
        pl.kernel,
        mesh=plsc.VectorSubcoreMesh(core_axis_name="c", subcore_axis_name="s"),
        out_type=jax.ShapeDtypeStruct((n, _EMB), jnp.float32),
        scratch_types=(
            [pltpu.VMEM((n_chunks, _C), jnp.int32)]
            + [pltpu.VMEM((_C, _EMB), jnp.float32) for _ in range(2)]
            + [pltpu.SemaphoreType.DMA for _ in range(4)]
        ),
    )(_emb_body)

    out = grid_kernel(tok3, table)
    return out.reshape(b, l, _EMB)


# speedup vs baseline: 3.9844x; 1.4585x over previous
import functools
import math

import jax
import jax.numpy as jnp
from jax import lax
from jax.experimental import pallas as pl
from jax.experimental.pallas import tpu as pltpu
from jax.experimental.pallas import tpu_sc as plsc

_EMB = 128
_SCALE = math.sqrt(_EMB)
_NC = 2
_NS = 16
_NW = _NC * _NS
_LANES = 16


def _emb_body(tok_hbm, table_hbm, out_hbm, idx_v, rows0, rows1, s_in0, s_in1,
              s_out0, s_out1):
    wid = lax.axis_index("s") * _NC + lax.axis_index("c")
    nb, seq = idx_v.shape
    b0 = wid * nb

    pltpu.sync_copy(tok_hbm.at[pl.ds(b0, nb)], idx_v)

    def scale(rows):
        def row(i, c):
            for j in range(_EMB // _LANES):
                sl = pl.ds(j * _LANES, _LANES)
                rows[i, sl] = rows[i, sl] * _SCALE
            return c

        lax.fori_loop(0, seq, row, 0)

    def outer(t, carry):
        a = 2 * t
        ga = pltpu.async_copy(table_hbm.at[idx_v.at[a]], rows0, s_in0)
        gb = pltpu.async_copy(table_hbm.at[idx_v.at[a + 1]], rows1, s_in1)
        ga.wait()
        scale(rows0)
        oa = pltpu.async_copy(rows0, out_hbm.at[b0 + a], s_out0)
        gb.wait()
        scale(rows1)
        ob = pltpu.async_copy(rows1, out_hbm.at[b0 + a + 1], s_out1)
        oa.wait()
        ob.wait()
        return carry

    lax.fori_loop(0, nb // 2, outer, 0)


def kernel(tokens, table):
    b, l = tokens.shape
    assert b % (2 * _NW) == 0
    nb = b // _NW
    tok = tokens.astype(jnp.int32)

    grid_kernel = functools.partial(
        pl.kernel,
        mesh=plsc.VectorSubcoreMesh(core_axis_name="c", subcore_axis_name="s"),
        out_type=jax.ShapeDtypeStruct((b, l, _EMB), jnp.float32),
        scratch_types=(
            [pltpu.VMEM((nb, l), jnp.int32)]
            + [pltpu.VMEM((l, _EMB), jnp.float32) for _ in range(2)]
            + [pltpu.SemaphoreType.DMA for _ in range(4)]
        ),
    )(_emb_body)

    return grid_kernel(tok, table)


# speedup vs baseline: 7.2937x; 1.8306x over previous
import functools
import math

import jax
import jax.numpy as jnp
from jax import lax
from jax.experimental import pallas as pl
from jax.experimental.pallas import tpu as pltpu
from jax.experimental.pallas import tpu_sc as plsc

_EMB = 128
_SCALE = math.sqrt(_EMB)
_NC = 2
_NS = 16
_NW = _NC * _NS
_LANES = 16


def _emb_body(tok_hbm, table_hbm, out_hbm, idx_v, rows0, rows1, s_in0, s_in1,
              s_out0, s_out1):
    wid = lax.axis_index("s") * _NC + lax.axis_index("c")
    seq, nb = idx_v.shape
    b0 = wid * nb

    pltpu.sync_copy(tok_hbm.at[:, pl.ds(b0, nb)], idx_v)

    def scale(rows):
        def row(i, c):
            for j in range(_EMB // _LANES):
                sl = pl.ds(j * _LANES, _LANES)
                rows[i, sl] = rows[i, sl] * _SCALE
            return c

        lax.fori_loop(0, nb, row, 0)

    def outer(t, carry):
        a = 2 * t
        ga = pltpu.async_copy(table_hbm.at[idx_v.at[a]], rows0, s_in0)
        gb = pltpu.async_copy(table_hbm.at[idx_v.at[a + 1]], rows1, s_in1)
        ga.wait()
        scale(rows0)
        oa = pltpu.async_copy(rows0, out_hbm.at[a, pl.ds(b0, nb)], s_out0)
        gb.wait()
        scale(rows1)
        ob = pltpu.async_copy(rows1, out_hbm.at[a + 1, pl.ds(b0, nb)], s_out1)
        oa.wait()
        ob.wait()
        return carry

    lax.fori_loop(0, seq // 2, outer, 0)


def kernel(tokens, table):
    b, l = tokens.shape
    assert b % _NW == 0 and l % 2 == 0
    nb = b // _NW
    tok_t = tokens.T.astype(jnp.int32)

    grid_kernel = functools.partial(
        pl.kernel,
        mesh=plsc.VectorSubcoreMesh(core_axis_name="c", subcore_axis_name="s"),
        out_type=jax.ShapeDtypeStruct((l, b, _EMB), jnp.float32),
        scratch_types=(
            [pltpu.VMEM((l, nb), jnp.int32)]
            + [pltpu.VMEM((nb, _EMB), jnp.float32) for _ in range(2)]
            + [pltpu.SemaphoreType.DMA for _ in range(4)]
        ),
    )(_emb_body)

    out = grid_kernel(tok_t, table)
    return jnp.transpose(out, (1, 0, 2))


# speedup vs baseline: 9.3814x; 1.2862x over previous
import functools
import math

import jax
import jax.numpy as jnp
from jax import lax
from jax.experimental import pallas as pl
from jax.experimental.pallas import tpu as pltpu
from jax.experimental.pallas import tpu_sc as plsc

_EMB = 128
_SCALE = math.sqrt(_EMB)
_NC = 2
_NS = 16
_NW = _NC * _NS
_LANES = 16


_K = 4


def _emb_body(tok_hbm, table_hbm, out_hbm, idx_v, *scratch):
    rows = list(scratch[:_K])
    s_in = list(scratch[_K:2 * _K])
    s_out = list(scratch[2 * _K:3 * _K])
    wid = lax.axis_index("s") * _NC + lax.axis_index("c")
    seq, nb = idx_v.shape
    b0 = wid * nb

    pltpu.sync_copy(tok_hbm.at[:, pl.ds(b0, nb)], idx_v)

    def scale(r):
        def row(i, c):
            for j in range(_EMB // _LANES):
                sl = pl.ds(j * _LANES, _LANES)
                r[i, sl] = r[i, sl] * _SCALE
            return c

        lax.fori_loop(0, nb, row, 0)

    def gather(h, b):
        pltpu.async_copy(table_hbm.at[idx_v.at[h]], rows[b], s_in[b])

    def slot(g, b, drain, issue):
        b2 = (b + 2) % _K
        if drain:
            pltpu.make_async_copy(
                rows[b2], out_hbm.at[g, pl.ds(b0, nb)], s_out[b2]
            ).wait()
        if issue:
            gather(g + 2, b2)
        pltpu.make_async_copy(table_hbm.at[idx_v.at[g]], rows[b], s_in[b]).wait()
        scale(rows[b])
        pltpu.async_copy(rows[b], out_hbm.at[g, pl.ds(b0, nb)], s_out[b])

    gather(0, 0)
    gather(1, 1)
    slot(0, 0, False, True)
    slot(1, 1, False, True)
    slot(2, 2, True, True)
    slot(3, 3, True, True)

    def outer(u, carry):
        g = 4 * u
        for b in range(_K):
            slot(g + b, b, True, True)
        return carry

    lax.fori_loop(1, (seq - 2) // 4, outer, 0)

    slot(seq - 2, (seq - 2) % _K, True, False)
    slot(seq - 1, (seq - 1) % _K, True, False)
    for g in (seq - 2, seq - 1):
        b = g % _K
        pltpu.make_async_copy(rows[b], out_hbm.at[g, pl.ds(b0, nb)], s_out[b]).wait()


def kernel(tokens, table):
    b, l = tokens.shape
    assert b % _NW == 0 and l >= 6 and (l - 2) % 4 == 0
    nb = b // _NW
    tok_t = tokens.T.astype(jnp.int32)

    grid_kernel = functools.partial(
        pl.kernel,
        mesh=plsc.VectorSubcoreMesh(core_axis_name="c", subcore_axis_name="s"),
        out_type=jax.ShapeDtypeStruct((l, b, _EMB), jnp.float32),
        scratch_types=(
            [pltpu.VMEM((l, nb), jnp.int32)]
            + [pltpu.VMEM((nb, _EMB), jnp.float32) for _ in range(_K)]
            + [pltpu.SemaphoreType.DMA for _ in range(2 * _K)]
        ),
    )(_emb_body)

    out = grid_kernel(tok_t, table)
    return jnp.transpose(out, (1, 0, 2))
